# baseline (speedup 1.0000x reference)
import functools

import jax
import jax.numpy as jnp
from jax import lax
from jax.experimental import pallas as pl
from jax.experimental.pallas import tpu as pltpu
from jax.experimental.pallas import tpu_sc as plsc

ALPHA_THRESHOLD = 1.0 / 255.0
MAX_ALPHA = 0.99
EPS = 1e-8
PIX_OFF = 0.5
H = 128
W = 128
G = 2048
KC = 16
NSTRIP = 16
SH = H // NSTRIP
NSEG = 2
NWORK = NSTRIP * NSEG
SEGG = G // NSEG
CAP = SEGG
NC = 2
LANES = 16
GB = CAP // 128


NEG_BIG = -1e30


def _tc_prep_body(pt_ref, prep_ref):
    mx = pt_ref[0:1, :]
    my = pt_ref[1:2, :]
    a = pt_ref[2:3, :]
    b = pt_ref[3:4, :]
    c = pt_ref[4:5, :]
    op = pt_ref[5:6, :]
    det = a * c - b * b
    valid = (op > ALPHA_THRESHOLD) & (det > EPS) & (a > 0.0) & (c > 0.0)
    tau = -2.0 * jnp.log(jnp.maximum(ALPHA_THRESHOLD / jnp.maximum(op, EPS), EPS))
    valid = valid & (tau > 0.0)
    wq = jnp.where(valid, det / jnp.maximum(a, EPS), 0.0)
    tau_b = jnp.where(valid, tau * 1.001 + 1e-5, -1.0)
    lnop = jnp.where(valid, jnp.log(jnp.maximum(op, EPS)), NEG_BIG)
    prep_ref[0:1, :] = -0.5 * a
    prep_ref[1:2, :] = -0.5 * (-2.0 * a * mx - 2.0 * b * my)
    prep_ref[2:3, :] = -0.5 * (2.0 * b)
    prep_ref[3:4, :] = -0.5 * (a * mx * mx + 2.0 * b * mx * my + c * my * my) + lnop
    prep_ref[4:5, :] = -0.5 * (-2.0 * b * mx - 2.0 * c * my)
    prep_ref[5:6, :] = -0.5 * c
    prep_ref[6:9, :] = pt_ref[6:9, :]
    prep_ref[9:16, :] = jnp.zeros((7, G), jnp.float32)
    prep_ref[16:17, :] = wq
    prep_ref[17:18, :] = tau_b
    prep_ref[18:19, :] = my
    prep_ref[19:24, :] = jnp.zeros((5, G), jnp.float32)


def _sc_bin_body(prep_h, params_h, gp_h, counts_h,
                 my_v, wq_v, tau_v, idx_v, rows_v, cnt_v, sem):
    wid = lax.axis_index("s") * NC + lax.axis_index("c")
    strip = wid // NSEG
    seg = wid % NSEG

    base = seg * SEGG
    pltpu.sync_copy(prep_h.at[18, pl.ds(base, SEGG)], my_v)
    pltpu.sync_copy(prep_h.at[16, pl.ds(base, SEGG)], wq_v)
    pltpu.sync_copy(prep_h.at[17, pl.ds(base, SEGG)], tau_v)

    ylo_c = strip.astype(jnp.float32) * float(SH) + PIX_OFF
    yhi_c = ylo_c + float(SH - 1)

    def zero_body(i, _):
        idx_v[i // 8, pl.ds((i % 8) * LANES, LANES)] = jnp.full((LANES,), G, jnp.int32)
        return 0

    lax.fori_loop(0, CAP // LANES, zero_body, 0)

    lane = lax.iota(jnp.int32, LANES) + base

    def scan_body(i, cnt):
        sl = pl.ds(i * LANES, LANES)
        myv = my_v[sl]
        wqv = wq_v[sl]
        tauv = tau_v[sl]
        dy = jnp.clip(myv, ylo_c, yhi_c) - myv
        m = (dy * dy) * wqv <= tauv
        pos = cnt + plsc.cumsum(m.astype(jnp.int32)) - 1
        ids = lane + i * LANES
        plsc.store_scatter(idx_v, [lax.div(pos, 128), lax.rem(pos, 128)], ids, mask=m)
        return cnt + plsc.all_reduce_population_count(m)

    cnt = lax.fori_loop(0, SEGG // LANES, scan_body, jnp.zeros((LANES,), jnp.int32))
    cnt_v[...] = cnt
    count = jnp.max(cnt)
    pltpu.sync_copy(cnt_v, counts_h.at[wid])

    for j in range(GB):
        @pl.when(count > j * 128)
        def _gather(j=j):
            pltpu.async_copy(params_h.at[idx_v.at[j]], rows_v.at[j], sem).wait()
            pltpu.sync_copy(rows_v.at[j], gp_h.at[wid, j])


_sc_bin = functools.partial(
    pl.kernel,
    out_type=(
        jax.ShapeDtypeStruct((NWORK, GB, 128, 16), jnp.float32),
        jax.ShapeDtypeStruct((NWORK, LANES), jnp.int32),
    ),
    mesh=plsc.VectorSubcoreMesh(core_axis_name="c", subcore_axis_name="s"),
    compiler_params=pltpu.CompilerParams(
        needs_layout_passes=False, use_tc_tiling_on_sc=False),
    scratch_types=[
        pltpu.VMEM((SEGG,), jnp.float32),
        pltpu.VMEM((SEGG,), jnp.float32),
        pltpu.VMEM((SEGG,), jnp.float32),
        pltpu.VMEM((GB, 128), jnp.int32),
        pltpu.VMEM((GB, 128, 16), jnp.float32),
        pltpu.VMEM((LANES,), jnp.int32),
        pltpu.SemaphoreType.DMA,
    ],
)(_sc_bin_body)


def _tc_comp_body(counts_ref, bg_ref, gps_ref, gp_ref, out_ref,
                  accr, accg, accb, trans_ref):
    i = pl.program_id(0)
    j = pl.program_id(1)
    strip = i // NSEG
    seg = lax.rem(i, NSEG)

    @pl.when((seg == 0) & (j == 0))
    def _init():
        accr[:, :] = jnp.zeros((SH, W), jnp.float32)
        accg[:, :] = jnp.zeros((SH, W), jnp.float32)
        accb[:, :] = jnp.zeros((SH, W), jnp.float32)
        trans_ref[:, :] = jnp.ones((SH, W), jnp.float32)

    count = counts_ref[i, 0]
    blockcnt = jnp.clip(count - j * 128, 0, 128)
    nch = lax.div(blockcnt + (KC - 1), KC)

    xsl = jax.lax.broadcasted_iota(jnp.int32, (1, W), 1).astype(jnp.float32) + PIX_OFF
    basis = jnp.concatenate(
        [xsl * xsl, xsl, jnp.ones((1, W), jnp.float32)], axis=0)
    y8 = (jax.lax.broadcasted_iota(jnp.int32, (1, SH), 1) + strip * SH
          ).astype(jnp.float32) + PIX_OFF

    def chunk(jc, carry):
        t, ar, ag_, ab_ = carry
        p = gp_ref[0, 0, pl.ds(jc * KC, KC), :]
        l0 = p[:, 0:1] + jnp.zeros((1, SH), jnp.float32)
        l1 = p[:, 1:2] + p[:, 2:3] * y8
        l2 = p[:, 3:4] + (p[:, 4:5] + p[:, 5:6] * y8) * y8
        lhs = jnp.stack([l0, l1, l2], axis=-1).reshape(KC * SH, 3)
        qm = jax.lax.dot_general(lhs, basis, (((1,), (0,)), ((), ())),
                                 precision=lax.Precision.HIGHEST,
                                 preferred_element_type=jnp.float32)
        araw = jnp.exp(qm)
        alpha = jnp.where(araw >= ALPHA_THRESHOLD,
                          jnp.minimum(araw, MAX_ALPHA), 0.0)

        for kq in range(KC // 4):
            gb4 = jc * KC + kq * 4
            a1 = alpha[(kq * 4 + 0) * SH:(kq * 4 + 1) * SH, :]
            a2 = alpha[(kq * 4 + 1) * SH:(kq * 4 + 2) * SH, :]
            a3 = alpha[(kq * 4 + 2) * SH:(kq * 4 + 3) * SH, :]
            a4 = alpha[(kq * 4 + 3) * SH:(kq * 4 + 4) * SH, :]
            u1 = 1.0 - a1
            u12 = u1 * (1.0 - a2)
            u123 = u12 * (1.0 - a3)
            w1 = t * a1
            w2 = t * (a2 * u1)
            w3 = t * (a3 * u12)
            w4 = t * (a4 * u123)
            t = t * (u123 * (1.0 - a4))
            for kk, w in enumerate((w1, w2, w3, w4)):
                g = gb4 + kk
                ar = ar + w * gps_ref[0, 0, g, 6]
                ag_ = ag_ + w * gps_ref[0, 0, g, 7]
                ab_ = ab_ + w * gps_ref[0, 0, g, 8]
        return t, ar, ag_, ab_

    t0 = (trans_ref[:, :], accr[:, :], accg[:, :], accb[:, :])
    t, ar, ag_, ab_ = lax.fori_loop(0, nch, chunk, t0)
    accr[:, :] = ar
    accg[:, :] = ag_
    accb[:, :] = ab_
    trans_ref[:, :] = t

    @pl.when((seg == NSEG - 1) & (j == GB - 1))
    def _fin():
        tt = trans_ref[:, :]
        out_ref[0, :, :] = accr[:, :] + tt * bg_ref[0]
        out_ref[1, :, :] = accg[:, :] + tt * bg_ref[1]
        out_ref[2, :, :] = accb[:, :] + tt * bg_ref[2]


def kernel(means2d, conics, colors, opacities, depths, background, image_height, image_width):
    order = jnp.argsort(lax.stop_gradient(depths))
    params = jnp.zeros((G, 16), jnp.float32)
    params = params.at[:, 0:2].set(means2d)
    params = params.at[:, 2:5].set(conics)
    params = params.at[:, 5].set(opacities)
    params = params.at[:, 6:9].set(colors)
    params = jnp.take(params, order, axis=0)

    prep = pl.pallas_call(
        _tc_prep_body,
        in_specs=[pl.BlockSpec((16, G), lambda: (0, 0))],
        out_specs=pl.BlockSpec((24, G), lambda: (0, 0)),
        out_shape=jax.ShapeDtypeStruct((24, G), jnp.float32),
    )(params.T)

    sentinel = jnp.zeros((8, 16), jnp.float32).at[:, 3].set(NEG_BIG)
    params_aug = jnp.concatenate([prep[0:16].T, sentinel], axis=0)
    gp, counts = _sc_bin(prep, params_aug)

    out = pl.pallas_call(
        _tc_comp_body,
        grid=(NWORK, GB),
        in_specs=[
            pl.BlockSpec(memory_space=pltpu.SMEM),
            pl.BlockSpec(memory_space=pltpu.SMEM),
            pl.BlockSpec((1, 1, 128, 16), lambda i, j: (i, j, 0, 0),
                         memory_space=pltpu.SMEM),
            pl.BlockSpec((1, 1, 128, 16), lambda i, j: (i, j, 0, 0)),
        ],
        out_specs=pl.BlockSpec((3, SH, W), lambda i, j: (0, i // NSEG, 0)),
        out_shape=jax.ShapeDtypeStruct((3, H, W), jnp.float32),
        scratch_shapes=[
            pltpu.VMEM((SH, W), jnp.float32),
            pltpu.VMEM((SH, W), jnp.float32),
            pltpu.VMEM((SH, W), jnp.float32),
            pltpu.VMEM((SH, W), jnp.float32),
        ],
    )(counts, background.astype(jnp.float32), gp, gp)
    return jnp.transpose(out, (1, 2, 0)).astype(means2d.dtype)


# speedup vs baseline: 1.0521x; 1.0521x over previous
import functools

import jax
import jax.numpy as jnp
from jax import lax
from jax.experimental import pallas as pl
from jax.experimental.pallas import tpu as pltpu
from jax.experimental.pallas import tpu_sc as plsc

ALPHA_THRESHOLD = 1.0 / 255.0
MAX_ALPHA = 0.99
EPS = 1e-8
PIX_OFF = 0.5
H = 128
W = 128
G = 2048
KC = 16
NSTRIP = 16
SH = H // NSTRIP
NSEG = 2
NWORK = NSTRIP * NSEG
SEGG = G // NSEG
CAP = SEGG
NC = 2
LANES = 16
GB = CAP // 128


NEG_BIG = -1e30


def _tc_prep_body(pt_ref, prep_ref):
    mx = pt_ref[0:1, :]
    my = pt_ref[1:2, :]
    a = pt_ref[2:3, :]
    b = pt_ref[3:4, :]
    c = pt_ref[4:5, :]
    op = pt_ref[5:6, :]
    det = a * c - b * b
    valid = (op > ALPHA_THRESHOLD) & (det > EPS) & (a > 0.0) & (c > 0.0)
    tau = -2.0 * jnp.log(jnp.maximum(ALPHA_THRESHOLD / jnp.maximum(op, EPS), EPS))
    valid = valid & (tau > 0.0)
    wq = jnp.where(valid, det / jnp.maximum(a, EPS), 0.0)
    tau_b = jnp.where(valid, tau * 1.001 + 1e-5, -1.0)
    sqa = jnp.sqrt(jnp.maximum(a, 0.0))
    rb = b / jnp.maximum(sqa, EPS)
    prep_ref[0:1, :] = mx
    prep_ref[1:2, :] = my
    prep_ref[2:3, :] = jnp.where(valid, sqa, 0.0)
    prep_ref[3:4, :] = jnp.where(valid, rb, 0.0)
    prep_ref[4:5, :] = wq
    prep_ref[5:6, :] = op
    prep_ref[6:9, :] = pt_ref[6:9, :]
    prep_ref[9:10, :] = jnp.where(valid, tau, -1.0)
    prep_ref[10:16, :] = jnp.zeros((6, G), jnp.float32)
    prep_ref[16:17, :] = wq
    prep_ref[17:18, :] = tau_b
    prep_ref[18:24, :] = jnp.zeros((6, G), jnp.float32)


def _sc_bin_body(prep_h, params_h, gp_h, counts_h,
                 my_v, wq_v, tau_v, idx_v, rows_v, cnt_v, sem):
    wid = lax.axis_index("s") * NC + lax.axis_index("c")
    strip = wid // NSEG
    seg = wid % NSEG

    base = seg * SEGG
    pltpu.sync_copy(prep_h.at[1, pl.ds(base, SEGG)], my_v)
    pltpu.sync_copy(prep_h.at[16, pl.ds(base, SEGG)], wq_v)
    pltpu.sync_copy(prep_h.at[17, pl.ds(base, SEGG)], tau_v)

    ylo_c = strip.astype(jnp.float32) * float(SH) + PIX_OFF
    yhi_c = ylo_c + float(SH - 1)

    def zero_body(i, _):
        idx_v[i // 8, pl.ds((i % 8) * LANES, LANES)] = jnp.full((LANES,), G, jnp.int32)
        return 0

    lax.fori_loop(0, CAP // LANES, zero_body, 0)

    lane = lax.iota(jnp.int32, LANES) + base

    def scan_body(i, cnt):
        sl = pl.ds(i * LANES, LANES)
        myv = my_v[sl]
        wqv = wq_v[sl]
        tauv = tau_v[sl]
        dy = jnp.clip(myv, ylo_c, yhi_c) - myv
        m = (dy * dy) * wqv <= tauv
        pos = cnt + plsc.cumsum(m.astype(jnp.int32)) - 1
        ids = lane + i * LANES
        plsc.store_scatter(idx_v, [lax.div(pos, 128), lax.rem(pos, 128)], ids, mask=m)
        return cnt + plsc.all_reduce_population_count(m)

    cnt = lax.fori_loop(0, SEGG // LANES, scan_body, jnp.zeros((LANES,), jnp.int32))
    cnt_v[...] = cnt
    count = jnp.max(cnt)
    pltpu.sync_copy(cnt_v, counts_h.at[wid])

    for j in range(GB):
        @pl.when(count > j * 128)
        def _gather(j=j):
            pltpu.async_copy(params_h.at[idx_v.at[j]], rows_v.at[j], sem).wait()
            pltpu.sync_copy(rows_v.at[j], gp_h.at[wid, j])


_sc_bin = functools.partial(
    pl.kernel,
    out_type=(
        jax.ShapeDtypeStruct((NWORK, GB, 128, 16), jnp.float32),
        jax.ShapeDtypeStruct((NWORK, LANES), jnp.int32),
    ),
    mesh=plsc.VectorSubcoreMesh(core_axis_name="c", subcore_axis_name="s"),
    compiler_params=pltpu.CompilerParams(
        needs_layout_passes=False, use_tc_tiling_on_sc=False),
    scratch_types=[
        pltpu.VMEM((SEGG,), jnp.float32),
        pltpu.VMEM((SEGG,), jnp.float32),
        pltpu.VMEM((SEGG,), jnp.float32),
        pltpu.VMEM((GB, 128), jnp.int32),
        pltpu.VMEM((GB, 128, 16), jnp.float32),
        pltpu.VMEM((LANES,), jnp.int32),
        pltpu.SemaphoreType.DMA,
    ],
)(_sc_bin_body)


def _tc_comp_body(counts_ref, bg_ref, gps_ref, gp_ref, out_ref,
                  accr, accg, accb, trans_ref):
    i = pl.program_id(0)
    j = pl.program_id(1)
    strip = i // NSEG
    seg = lax.rem(i, NSEG)

    @pl.when((seg == 0) & (j == 0))
    def _init():
        accr[:, :] = jnp.zeros((SH, W), jnp.float32)
        accg[:, :] = jnp.zeros((SH, W), jnp.float32)
        accb[:, :] = jnp.zeros((SH, W), jnp.float32)
        trans_ref[:, :] = jnp.ones((SH, W), jnp.float32)

    count = counts_ref[i, 0]
    blockcnt = jnp.clip(count - j * 128, 0, 128)
    nch = lax.div(blockcnt + (KC - 1), KC)

    xs = jax.lax.broadcasted_iota(jnp.int32, (1, 1, W), 2).astype(jnp.float32) + PIX_OFF
    ys = (jax.lax.broadcasted_iota(jnp.int32, (1, SH, 1), 1) + strip * SH
          ).astype(jnp.float32) + PIX_OFF

    def chunk(jc, carry):
        t, ar, ag_, ab_ = carry
        p = gp_ref[0, 0, pl.ds(jc * KC, KC), :]
        mx = p[:, 0:1][:, :, None]
        my = p[:, 1:2][:, :, None]
        sqa = p[:, 2:3][:, :, None]
        rb = p[:, 3:4][:, :, None]
        wq = p[:, 4:5][:, :, None]
        op = p[:, 5:6][:, :, None]
        tau = p[:, 9:10][:, :, None]

        dx = xs - mx
        dy = ys - my
        u = sqa * dx + rb * dy
        q = u * u + wq * (dy * dy)
        alpha = jnp.where(q <= tau, op * jnp.exp(-0.5 * q), 0.0)
        alpha = jnp.minimum(alpha, MAX_ALPHA)

        for kq in range(KC // 4):
            gb4 = jc * KC + kq * 4
            a1 = alpha[kq * 4 + 0]
            a2 = alpha[kq * 4 + 1]
            a3 = alpha[kq * 4 + 2]
            a4 = alpha[kq * 4 + 3]
            u1 = 1.0 - a1
            u12 = u1 * (1.0 - a2)
            u123 = u12 * (1.0 - a3)
            w1 = t * a1
            w2 = t * (a2 * u1)
            w3 = t * (a3 * u12)
            w4 = t * (a4 * u123)
            t = t * (u123 * (1.0 - a4))
            for kk, w in enumerate((w1, w2, w3, w4)):
                g = gb4 + kk
                ar = ar + w * gps_ref[0, 0, g, 6]
                ag_ = ag_ + w * gps_ref[0, 0, g, 7]
                ab_ = ab_ + w * gps_ref[0, 0, g, 8]
        return t, ar, ag_, ab_

    t0 = (trans_ref[:, :], accr[:, :], accg[:, :], accb[:, :])
    t, ar, ag_, ab_ = lax.fori_loop(0, nch, chunk, t0)
    accr[:, :] = ar
    accg[:, :] = ag_
    accb[:, :] = ab_
    trans_ref[:, :] = t

    @pl.when((seg == NSEG - 1) & (j == GB - 1))
    def _fin():
        tt = trans_ref[:, :]
        out_ref[0, :, :] = accr[:, :] + tt * bg_ref[0]
        out_ref[1, :, :] = accg[:, :] + tt * bg_ref[1]
        out_ref[2, :, :] = accb[:, :] + tt * bg_ref[2]


def kernel(means2d, conics, colors, opacities, depths, background, image_height, image_width):
    order = jnp.argsort(lax.stop_gradient(depths))
    params = jnp.zeros((G, 16), jnp.float32)
    params = params.at[:, 0:2].set(means2d)
    params = params.at[:, 2:5].set(conics)
    params = params.at[:, 5].set(opacities)
    params = params.at[:, 6:9].set(colors)
    params = jnp.take(params, order, axis=0)

    prep = pl.pallas_call(
        _tc_prep_body,
        in_specs=[pl.BlockSpec((16, G), lambda: (0, 0))],
        out_specs=pl.BlockSpec((24, G), lambda: (0, 0)),
        out_shape=jax.ShapeDtypeStruct((24, G), jnp.float32),
    )(params.T)

    params_aug = jnp.concatenate(
        [prep[0:16].T, jnp.zeros((8, 16), jnp.float32)], axis=0)
    gp, counts = _sc_bin(prep, params_aug)

    out = pl.pallas_call(
        _tc_comp_body,
        grid=(NWORK, GB),
        in_specs=[
            pl.BlockSpec(memory_space=pltpu.SMEM),
            pl.BlockSpec(memory_space=pltpu.SMEM),
            pl.BlockSpec((1, 1, 128, 16), lambda i, j: (i, j, 0, 0),
                         memory_space=pltpu.SMEM),
            pl.BlockSpec((1, 1, 128, 16), lambda i, j: (i, j, 0, 0)),
        ],
        out_specs=pl.BlockSpec((3, SH, W), lambda i, j: (0, i // NSEG, 0)),
        out_shape=jax.ShapeDtypeStruct((3, H, W), jnp.float32),
        scratch_shapes=[
            pltpu.VMEM((SH, W), jnp.float32),
            pltpu.VMEM((SH, W), jnp.float32),
            pltpu.VMEM((SH, W), jnp.float32),
            pltpu.VMEM((SH, W), jnp.float32),
        ],
    )(counts, background.astype(jnp.float32), gp, gp)
    return jnp.transpose(out, (1, 2, 0)).astype(means2d.dtype)


# speedup vs baseline: 1.9840x; 1.8856x over previous
import functools

import jax
import jax.numpy as jnp
from jax import lax
from jax.experimental import pallas as pl
from jax.experimental.pallas import tpu as pltpu
from jax.experimental.pallas import tpu_sc as plsc

ALPHA_THRESHOLD = 1.0 / 255.0
MAX_ALPHA = 0.99
EPS = 1e-8
PIX_OFF = 0.5
H = 128
W = 128
G = 2048
KC = 16
NSTRIP = 16
SH = H // NSTRIP
NSEG = 2
NWORK = NSTRIP * NSEG
SEGG = G // NSEG
CAP = SEGG
NC = 2
LANES = 16
GB = CAP // 128


NEG_BIG = -1e30


def _tc_prep_body(pt_ref, prep_ref):
    mx = pt_ref[0:1, :]
    my = pt_ref[1:2, :]
    a = pt_ref[2:3, :]
    b = pt_ref[3:4, :]
    c = pt_ref[4:5, :]
    op = pt_ref[5:6, :]
    det = a * c - b * b
    valid = (op > ALPHA_THRESHOLD) & (det > EPS) & (a > 0.0) & (c > 0.0)
    tau = -2.0 * jnp.log(jnp.maximum(ALPHA_THRESHOLD / jnp.maximum(op, EPS), EPS))
    valid = valid & (tau > 0.0)
    wq = jnp.where(valid, det / jnp.maximum(a, EPS), 0.0)
    tau_b = jnp.where(valid, tau * 1.001 + 1e-5, -1.0)
    sqa = jnp.sqrt(jnp.maximum(a, 0.0))
    rb = b / jnp.maximum(sqa, EPS)
    prep_ref[0:1, :] = mx
    prep_ref[1:2, :] = my
    prep_ref[2:3, :] = jnp.where(valid, sqa, 0.0)
    prep_ref[3:4, :] = jnp.where(valid, rb, 0.0)
    prep_ref[4:5, :] = wq
    prep_ref[5:6, :] = op
    prep_ref[6:9, :] = pt_ref[6:9, :]
    prep_ref[9:10, :] = jnp.where(valid, tau, -1.0)
    prep_ref[10:16, :] = jnp.zeros((6, G), jnp.float32)
    prep_ref[16:17, :] = wq
    prep_ref[17:18, :] = tau_b
    prep_ref[18:24, :] = jnp.zeros((6, G), jnp.float32)


def _sc_bin_body(prep_h, params_h, gp_h, counts_h,
                 my_v, wq_v, tau_v, idx_v, rows_v, cnt_v, sem):
    wid = lax.axis_index("s") * NC + lax.axis_index("c")
    strip = wid // NSEG
    seg = wid % NSEG

    base = seg * SEGG
    pltpu.sync_copy(prep_h.at[1, pl.ds(base, SEGG)], my_v)
    pltpu.sync_copy(prep_h.at[16, pl.ds(base, SEGG)], wq_v)
    pltpu.sync_copy(prep_h.at[17, pl.ds(base, SEGG)], tau_v)

    ylo_c = strip.astype(jnp.float32) * float(SH) + PIX_OFF
    yhi_c = ylo_c + float(SH - 1)

    def zero_body(i, _):
        idx_v[i // 8, pl.ds((i % 8) * LANES, LANES)] = jnp.full((LANES,), G, jnp.int32)
        return 0

    lax.fori_loop(0, CAP // LANES, zero_body, 0)

    lane = lax.iota(jnp.int32, LANES) + base

    def scan_body(i, cnt):
        sl = pl.ds(i * LANES, LANES)
        myv = my_v[sl]
        wqv = wq_v[sl]
        tauv = tau_v[sl]
        dy = jnp.clip(myv, ylo_c, yhi_c) - myv
        m = (dy * dy) * wqv <= tauv
        pos = cnt + plsc.cumsum(m.astype(jnp.int32)) - 1
        ids = lane + i * LANES
        plsc.store_scatter(idx_v, [lax.div(pos, 128), lax.rem(pos, 128)], ids, mask=m)
        return cnt + plsc.all_reduce_population_count(m)

    cnt = lax.fori_loop(0, SEGG // LANES, scan_body, jnp.zeros((LANES,), jnp.int32))
    cnt_v[...] = cnt
    count = jnp.max(cnt)
    pltpu.sync_copy(cnt_v, counts_h.at[wid])

    for j in range(GB):
        @pl.when(count > j * 128)
        def _gather(j=j):
            pltpu.async_copy(params_h.at[idx_v.at[j]], rows_v.at[j], sem).wait()
            pltpu.sync_copy(rows_v.at[j], gp_h.at[wid, j])


_sc_bin = functools.partial(
    pl.kernel,
    out_type=(
        jax.ShapeDtypeStruct((NWORK, GB, 128, 16), jnp.float32),
        jax.ShapeDtypeStruct((NWORK, LANES), jnp.int32),
    ),
    mesh=plsc.VectorSubcoreMesh(core_axis_name="c", subcore_axis_name="s"),
    compiler_params=pltpu.CompilerParams(
        needs_layout_passes=False, use_tc_tiling_on_sc=False),
    scratch_types=[
        pltpu.VMEM((SEGG,), jnp.float32),
        pltpu.VMEM((SEGG,), jnp.float32),
        pltpu.VMEM((SEGG,), jnp.float32),
        pltpu.VMEM((GB, 128), jnp.int32),
        pltpu.VMEM((GB, 128, 16), jnp.float32),
        pltpu.VMEM((LANES,), jnp.int32),
        pltpu.SemaphoreType.DMA,
    ],
)(_sc_bin_body)


def _tc_comp_body(counts_ref, bg_ref, gp_ref, out_ref,
                  accr, accg, accb, trans_ref):
    i = pl.program_id(0)
    strip = i // NSEG
    seg = lax.rem(i, NSEG)

    @pl.when(seg == 0)
    def _init():
        accr[:, :] = jnp.zeros((SH, W), jnp.float32)
        accg[:, :] = jnp.zeros((SH, W), jnp.float32)
        accb[:, :] = jnp.zeros((SH, W), jnp.float32)
        trans_ref[:, :] = jnp.ones((SH, W), jnp.float32)

    count = counts_ref[i, 0]
    nch = lax.div(count + (KC - 1), KC)

    xs = jax.lax.broadcasted_iota(jnp.int32, (1, 1, W), 2).astype(jnp.float32) + PIX_OFF
    ys = (jax.lax.broadcasted_iota(jnp.int32, (1, SH, 1), 1) + strip * SH
          ).astype(jnp.float32) + PIX_OFF

    def chunk(jc, carry):
        t, ar, ag_, ab_ = carry
        p = gp_ref[0, pl.ds(jc * KC, KC), :]
        mx = p[:, 0:1][:, :, None]
        my = p[:, 1:2][:, :, None]
        sqa = p[:, 2:3][:, :, None]
        rb = p[:, 3:4][:, :, None]
        wq = p[:, 4:5][:, :, None]
        op = p[:, 5:6][:, :, None]
        tau = p[:, 9:10][:, :, None]

        dx = xs - mx
        dy = ys - my
        u = sqa * dx + rb * dy
        q = u * u + wq * (dy * dy)
        alpha = jnp.where(q <= tau, op * jnp.exp(-0.5 * q), 0.0)
        alpha = jnp.minimum(alpha, MAX_ALPHA)

        cr = p[:, 6:7][:, :, None]
        cg = p[:, 7:8][:, :, None]
        cb = p[:, 8:9][:, :, None]
        for kq in range(KC // 4):
            a1 = alpha[kq * 4 + 0]
            a2 = alpha[kq * 4 + 1]
            a3 = alpha[kq * 4 + 2]
            a4 = alpha[kq * 4 + 3]
            u1 = 1.0 - a1
            u12 = u1 * (1.0 - a2)
            u123 = u12 * (1.0 - a3)
            w1 = t * a1
            w2 = t * (a2 * u1)
            w3 = t * (a3 * u12)
            w4 = t * (a4 * u123)
            t = t * (u123 * (1.0 - a4))
            for kk, w in enumerate((w1, w2, w3, w4)):
                g = kq * 4 + kk
                ar = ar + w * cr[g]
                ag_ = ag_ + w * cg[g]
                ab_ = ab_ + w * cb[g]
        return t, ar, ag_, ab_

    t0 = (trans_ref[:, :], accr[:, :], accg[:, :], accb[:, :])
    t, ar, ag_, ab_ = lax.fori_loop(0, nch, chunk, t0)
    accr[:, :] = ar
    accg[:, :] = ag_
    accb[:, :] = ab_
    trans_ref[:, :] = t

    @pl.when(seg == NSEG - 1)
    def _fin():
        tt = trans_ref[:, :]
        out_ref[0, :, :] = accr[:, :] + tt * bg_ref[0]
        out_ref[1, :, :] = accg[:, :] + tt * bg_ref[1]
        out_ref[2, :, :] = accb[:, :] + tt * bg_ref[2]


def kernel(means2d, conics, colors, opacities, depths, background, image_height, image_width):
    order = jnp.argsort(lax.stop_gradient(depths))
    params = jnp.zeros((G, 16), jnp.float32)
    params = params.at[:, 0:2].set(means2d)
    params = params.at[:, 2:5].set(conics)
    params = params.at[:, 5].set(opacities)
    params = params.at[:, 6:9].set(colors)
    params = jnp.take(params, order, axis=0)

    prep = pl.pallas_call(
        _tc_prep_body,
        in_specs=[pl.BlockSpec((16, G), lambda: (0, 0))],
        out_specs=pl.BlockSpec((24, G), lambda: (0, 0)),
        out_shape=jax.ShapeDtypeStruct((24, G), jnp.float32),
    )(params.T)

    params_aug = jnp.concatenate(
        [prep[0:16].T, jnp.zeros((8, 16), jnp.float32)], axis=0)
    gp, counts = _sc_bin(prep, params_aug)
    gp = gp.reshape(NWORK, CAP, 16)

    out = pl.pallas_call(
        _tc_comp_body,
        grid=(NWORK,),
        in_specs=[
            pl.BlockSpec(memory_space=pltpu.SMEM),
            pl.BlockSpec(memory_space=pltpu.SMEM),
            pl.BlockSpec((1, CAP, 16), lambda i: (i, 0, 0)),
        ],
        out_specs=pl.BlockSpec((3, SH, W), lambda i: (0, i // NSEG, 0)),
        out_shape=jax.ShapeDtypeStruct((3, H, W), jnp.float32),
        scratch_shapes=[
            pltpu.VMEM((SH, W), jnp.float32),
            pltpu.VMEM((SH, W), jnp.float32),
            pltpu.VMEM((SH, W), jnp.float32),
            pltpu.VMEM((SH, W), jnp.float32),
        ],
    )(counts, background.astype(jnp.float32), gp)
    return jnp.transpose(out, (1, 2, 0)).astype(means2d.dtype)
